# baseline (speedup 1.0000x reference)
import dataclasses
import functools

import jax
import jax.numpy as jnp
from jax import lax
from jax.experimental import pallas as pl
from jax.experimental.pallas import tpu as pltpu
from jax.experimental.pallas import tpu_sc as plsc

DIM = 1024
SEQ = 2048
NB = 2
TOPK = 4
BQ = 512
BQA = 256
NQB = SEQ // BQA

NEG_INF = float("-inf")


_DOT_T = functools.partial(
    lax.dot_general,
    dimension_numbers=(((1,), (1,)), ((), ())),
    preferred_element_type=jnp.float32,
    precision=lax.Precision.DEFAULT,
)


def _proj_body(x_ref, wk_ref, bk_ref, wv_ref, bv_ref, k_ref, v_ref):
    xb = x_ref[...]
    k_ref[...] = _DOT_T(xb, wk_ref[...]) + bk_ref[...]
    v_ref[...] = _DOT_T(xb, wv_ref[...]) + bv_ref[...]


def _proj_stage(x2d, Wk, bk, Wv, bv):
    nrb = (NB * SEQ) // BQ
    row = pl.BlockSpec((BQ, DIM), lambda i: (i, 0))
    full = pl.BlockSpec((DIM, DIM), lambda i: (0, 0))
    bias = pl.BlockSpec((1, DIM), lambda i: (0, 0))
    shp = jax.ShapeDtypeStruct((NB * SEQ, DIM), jnp.float32)
    return pl.pallas_call(
        _proj_body,
        grid=(nrb,),
        in_specs=[row, full, bias, full, bias],
        out_specs=[row, row],
        out_shape=[shp, shp],
    )(x2d, Wk, bk.reshape(1, DIM), Wv, bv.reshape(1, DIM))


def _attn_body(b_off, x_ref, wq_ref, bq_ref, k_ref, w_ref, idx_ref):
    q = _DOT_T(x_ref[...], wq_ref[...]) + bq_ref[...]
    sim = _DOT_T(q, k_ref[...])

    col = lax.broadcasted_iota(jnp.int32, (BQA, SEQ), 1).astype(jnp.float32)
    s = sim
    vals = []
    idxs = []
    for r in range(TOPK):
        m = jnp.max(s, axis=1, keepdims=True)
        hit = s == m
        idx = jnp.min(jnp.where(hit, col, float(SEQ)), axis=1, keepdims=True)
        if r + 1 < TOPK:
            s = jnp.where(col == idx, NEG_INF, s)
        vals.append(m)
        idxs.append(idx)

    ps = [jnp.exp(v - vals[0]) for v in vals]
    z = ps[0] + ps[1] + ps[2] + ps[3]
    w_ref[...] = jnp.concatenate(ps, axis=1) / z
    idx_ref[...] = (jnp.concatenate(idxs, axis=1).astype(jnp.int32) + b_off)


def _attn_stage(x2d, Wq, bq, k2d, b):
    return pl.pallas_call(
        functools.partial(_attn_body, b * SEQ),
        grid=(NQB,),
        in_specs=[
            pl.BlockSpec((BQA, DIM), lambda i: (b * (SEQ // BQA) + i, 0)),
            pl.BlockSpec((DIM, DIM), lambda i: (0, 0)),
            pl.BlockSpec((1, DIM), lambda i: (0, 0)),
            pl.BlockSpec((SEQ, DIM), lambda i: (b, 0)),
        ],
        out_specs=[
            pl.BlockSpec((BQA, TOPK), lambda i: (i, 0)),
            pl.BlockSpec((BQA, TOPK), lambda i: (i, 0)),
        ],
        out_shape=[
            jax.ShapeDtypeStruct((SEQ, TOPK), jnp.float32),
            jax.ShapeDtypeStruct((SEQ, TOPK), jnp.int32),
        ],
    )(x2d, Wq, bq.reshape(1, DIM), k2d)


NQ = NB * SEQ
NW = 32
QPW = SEQ // NW
CQ = 8
CR = CQ * TOPK
LANES = 16


NCHUNK = QPW // CQ


def _sc_body(v_hbm, idx_hbm, w_hbm, out_hbm, idx_v, w_v, rows_v, out_v, sems):
    wid = lax.axis_index("s") * 2 + lax.axis_index("c")
    qbase = wid * QPW

    def issue(c):
        p = c % 2
        ebase = (qbase + c * CQ) * TOPK
        pltpu.sync_copy(idx_hbm.at[pl.ds(ebase, CR)], idx_v.at[p])
        pltpu.sync_copy(w_hbm.at[pl.ds(ebase, CR)], w_v.at[p])
        return pltpu.async_copy(v_hbm.at[idx_v.at[p]], rows_v.at[p],
                                sems.at[p])

    cps = {0: issue(0)}
    for c in range(NCHUNK):
        p = c % 2
        if c + 1 < NCHUNK:
            cps[c + 1] = issue(c + 1)
        cps[c].wait()
        for qi in range(CQ):
            wb = [
                plsc.load_gather(
                    w_v.at[p], [jnp.full((LANES,), qi * TOPK + j, jnp.int32)])
                for j in range(TOPK)
            ]

            @pl.loop(0, DIM, step=LANES * 8)
            def _(d):
                for u in range(8):
                    sl = pl.ds(d + u * LANES, LANES)
                    acc = rows_v[p, qi * TOPK + 0, sl] * wb[0]
                    acc = acc + rows_v[p, qi * TOPK + 1, sl] * wb[1]
                    acc = acc + rows_v[p, qi * TOPK + 2, sl] * wb[2]
                    acc = acc + rows_v[p, qi * TOPK + 3, sl] * wb[3]
                    out_v[qi, sl] = acc

        pltpu.sync_copy(out_v, out_hbm.at[pl.ds(qbase + c * CQ, CQ)])


def _sc_stage(v_flat, idx_flat, w_flat):
    mesh = plsc.VectorSubcoreMesh(core_axis_name="c", subcore_axis_name="s")
    cp = pltpu.CompilerParams()
    if "needs_layout_passes" in pltpu.CompilerParams.__dataclass_fields__:
        cp = dataclasses.replace(cp, needs_layout_passes=False)
    f = pl.kernel(
        _sc_body,
        out_type=jax.ShapeDtypeStruct((SEQ, DIM), jnp.float32),
        mesh=mesh,
        scratch_types=[
            pltpu.VMEM((2, CR), jnp.int32),
            pltpu.VMEM((2, CR), jnp.float32),
            pltpu.VMEM((2, CR, DIM), jnp.float32),
            pltpu.VMEM((CQ, DIM), jnp.float32),
            pltpu.SemaphoreType.DMA((2,)),
        ],
        compiler_params=cp,
    )
    return f(v_flat, idx_flat, w_flat)


def kernel(x, Wq, bq, Wk, bk, Wv, bv):
    x2d = x.reshape(NB * SEQ, DIM)
    k, v = _proj_stage(x2d, Wk, bk, Wv, bv)
    wis = [_attn_stage(x2d, Wq, bq, k, b) for b in range(NB)]
    outs = [_sc_stage(v, idx.reshape(SEQ * TOPK), w.reshape(SEQ * TOPK))
            for (w, idx) in wis]
    return jnp.stack(outs).reshape(NB, SEQ, DIM)


# speedup vs baseline: 1.0153x; 1.0153x over previous
import dataclasses
import functools

import jax
import jax.numpy as jnp
from jax import lax
from jax.experimental import pallas as pl
from jax.experimental.pallas import tpu as pltpu
from jax.experimental.pallas import tpu_sc as plsc

DIM = 1024
SEQ = 2048
NB = 2
TOPK = 4
BQ = 512
BQA = 512
NQB = SEQ // BQA

NEG_INF = float("-inf")


_DOT_T = functools.partial(
    lax.dot_general,
    dimension_numbers=(((1,), (1,)), ((), ())),
    preferred_element_type=jnp.float32,
    precision=lax.Precision.DEFAULT,
)


def _proj_body(x_ref, wk_ref, bk_ref, wv_ref, bv_ref, k_ref, v_ref):
    xb = x_ref[...]
    k_ref[...] = _DOT_T(xb, wk_ref[...]) + bk_ref[...]
    v_ref[...] = _DOT_T(xb, wv_ref[...]) + bv_ref[...]


def _proj_stage(x2d, Wk, bk, Wv, bv):
    nrb = (NB * SEQ) // BQ
    row = pl.BlockSpec((BQ, DIM), lambda i: (i, 0))
    full = pl.BlockSpec((DIM, DIM), lambda i: (0, 0))
    bias = pl.BlockSpec((1, DIM), lambda i: (0, 0))
    shp = jax.ShapeDtypeStruct((NB * SEQ, DIM), jnp.float32)
    return pl.pallas_call(
        _proj_body,
        grid=(nrb,),
        in_specs=[row, full, bias, full, bias],
        out_specs=[row, row],
        out_shape=[shp, shp],
    )(x2d, Wk, bk.reshape(1, DIM), Wv, bv.reshape(1, DIM))


def _attn_body(b_off, x_ref, wq_ref, bq_ref, k_ref, w_ref, idx_ref):
    q = _DOT_T(x_ref[...], wq_ref[...]) + bq_ref[...]
    sim = _DOT_T(q, k_ref[...])

    half = SEQ // 2
    colL = lax.broadcasted_iota(jnp.int32, (BQA, half), 1).astype(jnp.float32)
    colR = colL + float(half)
    sl = sim[:, :half]
    sr = sim[:, half:]
    ge = sl >= sr
    a = jnp.maximum(sl, sr)
    b = jnp.minimum(sl, sr)
    ai = jnp.where(ge, colL, colR)
    bi = jnp.where(ge, colR, colL)
    vals = []
    idxs = []
    for r in range(TOPK):
        m = jnp.max(a, axis=1, keepdims=True)
        hit = a == m
        idx = jnp.min(jnp.where(hit, ai, float(SEQ)), axis=1, keepdims=True)
        vals.append(m)
        idxs.append(idx)
        if r + 1 < TOPK:
            pm = ai == idx
            a = jnp.where(pm, b, a)
            ai = jnp.where(pm, bi, ai)
            b = jnp.where(pm, NEG_INF, b)

    ps = [jnp.exp(v - vals[0]) for v in vals]
    z = ps[0] + ps[1] + ps[2] + ps[3]
    w_ref[...] = jnp.concatenate(ps, axis=1) / z
    idx_ref[...] = (jnp.concatenate(idxs, axis=1).astype(jnp.int32) + b_off)


def _attn_stage(x2d, Wq, bq, k2d, b):
    return pl.pallas_call(
        functools.partial(_attn_body, b * SEQ),
        grid=(NQB,),
        in_specs=[
            pl.BlockSpec((BQA, DIM), lambda i: (b * (SEQ // BQA) + i, 0)),
            pl.BlockSpec((DIM, DIM), lambda i: (0, 0)),
            pl.BlockSpec((1, DIM), lambda i: (0, 0)),
            pl.BlockSpec((SEQ, DIM), lambda i: (b, 0)),
        ],
        out_specs=[
            pl.BlockSpec((BQA, TOPK), lambda i: (i, 0)),
            pl.BlockSpec((BQA, TOPK), lambda i: (i, 0)),
        ],
        out_shape=[
            jax.ShapeDtypeStruct((SEQ, TOPK), jnp.float32),
            jax.ShapeDtypeStruct((SEQ, TOPK), jnp.int32),
        ],
    )(x2d, Wq, bq.reshape(1, DIM), k2d)


NQ = NB * SEQ
NW = 32
QPW = SEQ // NW
CQ = 8
CR = CQ * TOPK
LANES = 16


NCHUNK = QPW // CQ


def _sc_body(v_hbm, idx_hbm, w_hbm, out_hbm, idx_v, w_v, rows_v, out_v, sems):
    wid = lax.axis_index("s") * 2 + lax.axis_index("c")
    qbase = wid * QPW

    def issue(c):
        p = c % 2
        ebase = (qbase + c * CQ) * TOPK
        pltpu.sync_copy(idx_hbm.at[pl.ds(ebase, CR)], idx_v.at[p])
        pltpu.sync_copy(w_hbm.at[pl.ds(ebase, CR)], w_v.at[p])
        return pltpu.async_copy(v_hbm.at[idx_v.at[p]], rows_v.at[p],
                                sems.at[p])

    cps = {0: issue(0)}
    for c in range(NCHUNK):
        p = c % 2
        if c + 1 < NCHUNK:
            cps[c + 1] = issue(c + 1)
        cps[c].wait()
        for qi in range(CQ):
            wb = [
                plsc.load_gather(
                    w_v.at[p], [jnp.full((LANES,), qi * TOPK + j, jnp.int32)])
                for j in range(TOPK)
            ]

            @pl.loop(0, DIM, step=LANES * 8)
            def _(d):
                for u in range(8):
                    sl = pl.ds(d + u * LANES, LANES)
                    acc = rows_v[p, qi * TOPK + 0, sl] * wb[0]
                    acc = acc + rows_v[p, qi * TOPK + 1, sl] * wb[1]
                    acc = acc + rows_v[p, qi * TOPK + 2, sl] * wb[2]
                    acc = acc + rows_v[p, qi * TOPK + 3, sl] * wb[3]
                    out_v[qi, sl] = acc

        pltpu.sync_copy(out_v, out_hbm.at[pl.ds(qbase + c * CQ, CQ)])


def _sc_stage(v_flat, idx_flat, w_flat):
    mesh = plsc.VectorSubcoreMesh(core_axis_name="c", subcore_axis_name="s")
    cp = pltpu.CompilerParams()
    if "needs_layout_passes" in pltpu.CompilerParams.__dataclass_fields__:
        cp = dataclasses.replace(cp, needs_layout_passes=False)
    f = pl.kernel(
        _sc_body,
        out_type=jax.ShapeDtypeStruct((SEQ, DIM), jnp.float32),
        mesh=mesh,
        scratch_types=[
            pltpu.VMEM((2, CR), jnp.int32),
            pltpu.VMEM((2, CR), jnp.float32),
            pltpu.VMEM((2, CR, DIM), jnp.float32),
            pltpu.VMEM((CQ, DIM), jnp.float32),
            pltpu.SemaphoreType.DMA((2,)),
        ],
        compiler_params=cp,
    )
    return f(v_flat, idx_flat, w_flat)


def kernel(x, Wq, bq, Wk, bk, Wv, bv):
    x2d = x.reshape(NB * SEQ, DIM)
    k, v = _proj_stage(x2d, Wk, bk, Wv, bv)
    wis = [_attn_stage(x2d, Wq, bq, k, b) for b in range(NB)]
    outs = [_sc_stage(v, idx.reshape(SEQ * TOPK), w.reshape(SEQ * TOPK))
            for (w, idx) in wis]
    return jnp.stack(outs).reshape(NB, SEQ, DIM)


# speedup vs baseline: 1.0344x; 1.0188x over previous
import dataclasses
import functools

import jax
import jax.numpy as jnp
from jax import lax
from jax.experimental import pallas as pl
from jax.experimental.pallas import tpu as pltpu
from jax.experimental.pallas import tpu_sc as plsc

DIM = 1024
SEQ = 2048
NB = 2
TOPK = 4
BQ = 512
BQA = 512
NQB = SEQ // BQA

NEG_INF = float("-inf")


_DOT_T = functools.partial(
    lax.dot_general,
    dimension_numbers=(((1,), (1,)), ((), ())),
    preferred_element_type=jnp.float32,
    precision=lax.Precision.DEFAULT,
)


def _proj_body(x_ref, wk_ref, bk_ref, wv_ref, bv_ref, k_ref, v_ref):
    xb = x_ref[...]
    k_ref[...] = _DOT_T(xb, wk_ref[...]) + bk_ref[...]
    v_ref[...] = _DOT_T(xb, wv_ref[...]) + bv_ref[...]


def _proj_stage(x2d, Wk, bk, Wv, bv):
    nrb = (NB * SEQ) // BQ
    row = pl.BlockSpec((BQ, DIM), lambda i: (i, 0))
    full = pl.BlockSpec((DIM, DIM), lambda i: (0, 0))
    bias = pl.BlockSpec((1, DIM), lambda i: (0, 0))
    shp = jax.ShapeDtypeStruct((NB * SEQ, DIM), jnp.float32)
    return pl.pallas_call(
        _proj_body,
        grid=(nrb,),
        in_specs=[row, full, bias, full, bias],
        out_specs=[row, row],
        out_shape=[shp, shp],
    )(x2d, Wk, bk.reshape(1, DIM), Wv, bv.reshape(1, DIM))


def _attn_body(b_off, x_ref, wq_ref, bq_ref, k_ref, w_ref, idx_ref):
    q = _DOT_T(x_ref[...], wq_ref[...]) + bq_ref[...]
    sim = _DOT_T(q, k_ref[...])

    half = SEQ // 2
    colL = lax.broadcasted_iota(jnp.int32, (BQA, half), 1).astype(jnp.float32)
    colR = colL + float(half)
    sl = sim[:, :half]
    sr = sim[:, half:]
    ge = sl >= sr
    a = jnp.maximum(sl, sr)
    b = jnp.minimum(sl, sr)
    ai = jnp.where(ge, colL, colR)
    bi = jnp.where(ge, colR, colL)
    vals = []
    idxs = []
    for r in range(TOPK):
        m = jnp.max(a, axis=1, keepdims=True)
        hit = a == m
        idx = jnp.min(jnp.where(hit, ai, float(SEQ)), axis=1, keepdims=True)
        vals.append(m)
        idxs.append(idx)
        if r + 1 < TOPK:
            pm = ai == idx
            a = jnp.where(pm, b, a)
            ai = jnp.where(pm, bi, ai)
            b = jnp.where(pm, NEG_INF, b)

    ps = [jnp.exp(v - vals[0]) for v in vals]
    z = ps[0] + ps[1] + ps[2] + ps[3]
    w_ref[...] = jnp.concatenate(ps, axis=1) / z
    idx_ref[...] = (jnp.concatenate(idxs, axis=1).astype(jnp.int32) + b_off)


def _attn_stage(x2d, Wq, bq, k2d, b):
    return pl.pallas_call(
        functools.partial(_attn_body, b * SEQ),
        grid=(NQB,),
        in_specs=[
            pl.BlockSpec((BQA, DIM), lambda i: (b * (SEQ // BQA) + i, 0)),
            pl.BlockSpec((DIM, DIM), lambda i: (0, 0)),
            pl.BlockSpec((1, DIM), lambda i: (0, 0)),
            pl.BlockSpec((SEQ, DIM), lambda i: (b, 0)),
        ],
        out_specs=[
            pl.BlockSpec((BQA, TOPK), lambda i: (i, 0)),
            pl.BlockSpec((BQA, TOPK), lambda i: (i, 0)),
        ],
        out_shape=[
            jax.ShapeDtypeStruct((SEQ, TOPK), jnp.float32),
            jax.ShapeDtypeStruct((SEQ, TOPK), jnp.int32),
        ],
    )(x2d, Wq, bq.reshape(1, DIM), k2d)


NQ = NB * SEQ
NW = 32
QPW = SEQ // NW
CQ = 8
CR = CQ * TOPK
LANES = 16


NCHUNK = QPW // CQ


def _sc_body(v_hbm, idx_hbm, w_hbm, out_hbm, idx_v, w_v, rows_v, out_v,
             sems, osems):
    wid = lax.axis_index("s") * 2 + lax.axis_index("c")
    qbase = wid * QPW

    pltpu.sync_copy(idx_hbm.at[pl.ds(qbase * TOPK, QPW * TOPK)], idx_v)
    pltpu.sync_copy(w_hbm.at[pl.ds(qbase * TOPK, QPW * TOPK)], w_v)

    def issue(c):
        return pltpu.async_copy(v_hbm.at[idx_v.at[pl.ds(c * CR, CR)]],
                                rows_v.at[c % 2], sems.at[c % 2])

    cps = {0: issue(0)}
    ocps = {}
    for c in range(NCHUNK):
        p = c % 2
        if c + 1 < NCHUNK:
            cps[c + 1] = issue(c + 1)
        cps[c].wait()
        if c >= 2:
            ocps[c - 2].wait()
        for qi in range(CQ):
            wb = [
                plsc.load_gather(
                    w_v, [jnp.full((LANES,), c * CR + qi * TOPK + j,
                                   jnp.int32)])
                for j in range(TOPK)
            ]

            @pl.loop(0, DIM, step=LANES * 8)
            def _(d):
                for u in range(8):
                    sl = pl.ds(d + u * LANES, LANES)
                    acc = rows_v[p, qi * TOPK + 0, sl] * wb[0]
                    acc = acc + rows_v[p, qi * TOPK + 1, sl] * wb[1]
                    acc = acc + rows_v[p, qi * TOPK + 2, sl] * wb[2]
                    acc = acc + rows_v[p, qi * TOPK + 3, sl] * wb[3]
                    out_v[p, qi, sl] = acc

        ocps[c] = pltpu.async_copy(out_v.at[p],
                                   out_hbm.at[pl.ds(qbase + c * CQ, CQ)],
                                   osems.at[p])
    ocps[NCHUNK - 2].wait()
    ocps[NCHUNK - 1].wait()


def _sc_stage(v_flat, idx_flat, w_flat):
    mesh = plsc.VectorSubcoreMesh(core_axis_name="c", subcore_axis_name="s")
    cp = pltpu.CompilerParams()
    if "needs_layout_passes" in pltpu.CompilerParams.__dataclass_fields__:
        cp = dataclasses.replace(cp, needs_layout_passes=False)
    f = pl.kernel(
        _sc_body,
        out_type=jax.ShapeDtypeStruct((SEQ, DIM), jnp.float32),
        mesh=mesh,
        scratch_types=[
            pltpu.VMEM((QPW * TOPK,), jnp.int32),
            pltpu.VMEM((QPW * TOPK,), jnp.float32),
            pltpu.VMEM((2, CR, DIM), jnp.float32),
            pltpu.VMEM((2, CQ, DIM), jnp.float32),
            pltpu.SemaphoreType.DMA((2,)),
            pltpu.SemaphoreType.DMA((2,)),
        ],
        compiler_params=cp,
    )
    return f(v_flat, idx_flat, w_flat)


def kernel(x, Wq, bq, Wk, bk, Wv, bv):
    x2d = x.reshape(NB * SEQ, DIM)
    k, v = _proj_stage(x2d, Wk, bk, Wv, bv)
    wis = [_attn_stage(x2d, Wq, bq, k, b) for b in range(NB)]
    outs = [_sc_stage(v, idx.reshape(SEQ * TOPK), w.reshape(SEQ * TOPK))
            for (w, idx) in wis]
    return jnp.stack(outs).reshape(NB, SEQ, DIM)


# speedup vs baseline: 1.0376x; 1.0031x over previous
import dataclasses
import functools

import jax
import jax.numpy as jnp
from jax import lax
from jax.experimental import pallas as pl
from jax.experimental.pallas import tpu as pltpu
from jax.experimental.pallas import tpu_sc as plsc

DIM = 1024
SEQ = 2048
NB = 2
TOPK = 4
BQ = 1024
BQA = 512
NQB = SEQ // BQA

NEG_INF = float("-inf")


_DOT_T = functools.partial(
    lax.dot_general,
    dimension_numbers=(((1,), (1,)), ((), ())),
    preferred_element_type=jnp.float32,
    precision=lax.Precision.DEFAULT,
)


def _proj_body(x_ref, wk_ref, bk_ref, wv_ref, bv_ref, k_ref, v_ref):
    xb = x_ref[...]
    k_ref[...] = _DOT_T(xb, wk_ref[...]) + bk_ref[...]
    v_ref[...] = _DOT_T(xb, wv_ref[...]) + bv_ref[...]


def _proj_stage(x2d, Wk, bk, Wv, bv):
    nrb = (NB * SEQ) // BQ
    row = pl.BlockSpec((BQ, DIM), lambda i: (i, 0))
    full = pl.BlockSpec((DIM, DIM), lambda i: (0, 0))
    bias = pl.BlockSpec((1, DIM), lambda i: (0, 0))
    shp = jax.ShapeDtypeStruct((NB * SEQ, DIM), jnp.float32)
    return pl.pallas_call(
        _proj_body,
        grid=(nrb,),
        in_specs=[row, full, bias, full, bias],
        out_specs=[row, row],
        out_shape=[shp, shp],
    )(x2d, Wk, bk.reshape(1, DIM), Wv, bv.reshape(1, DIM))


def _attn_body(b_off, x_ref, wq_ref, bq_ref, k_ref, w_ref, idx_ref):
    q = _DOT_T(x_ref[...], wq_ref[...]) + bq_ref[...]
    sim = _DOT_T(q, k_ref[...])

    half = SEQ // 2
    colL = lax.broadcasted_iota(jnp.int32, (BQA, half), 1).astype(jnp.float32)
    colR = colL + float(half)
    sl = sim[:, :half]
    sr = sim[:, half:]
    ge = sl >= sr
    a = jnp.maximum(sl, sr)
    b = jnp.minimum(sl, sr)
    ai = jnp.where(ge, colL, colR)
    bi = jnp.where(ge, colR, colL)
    vals = []
    idxs = []
    for r in range(TOPK):
        m = jnp.max(a, axis=1, keepdims=True)
        hit = a == m
        idx = jnp.min(jnp.where(hit, ai, float(SEQ)), axis=1, keepdims=True)
        vals.append(m)
        idxs.append(idx)
        if r + 1 < TOPK:
            pm = ai == idx
            a = jnp.where(pm, b, a)
            ai = jnp.where(pm, bi, ai)
            b = jnp.where(pm, NEG_INF, b)

    ps = [jnp.exp(v - vals[0]) for v in vals]
    z = ps[0] + ps[1] + ps[2] + ps[3]
    w_ref[...] = jnp.concatenate(ps, axis=1) / z
    idx_ref[...] = (jnp.concatenate(idxs, axis=1).astype(jnp.int32) + b_off)


def _attn_stage(x2d, Wq, bq, k2d, b):
    return pl.pallas_call(
        functools.partial(_attn_body, b * SEQ),
        grid=(NQB,),
        in_specs=[
            pl.BlockSpec((BQA, DIM), lambda i: (b * (SEQ // BQA) + i, 0)),
            pl.BlockSpec((DIM, DIM), lambda i: (0, 0)),
            pl.BlockSpec((1, DIM), lambda i: (0, 0)),
            pl.BlockSpec((SEQ, DIM), lambda i: (b, 0)),
        ],
        out_specs=[
            pl.BlockSpec((BQA, TOPK), lambda i: (i, 0)),
            pl.BlockSpec((BQA, TOPK), lambda i: (i, 0)),
        ],
        out_shape=[
            jax.ShapeDtypeStruct((SEQ, TOPK), jnp.float32),
            jax.ShapeDtypeStruct((SEQ, TOPK), jnp.int32),
        ],
    )(x2d, Wq, bq.reshape(1, DIM), k2d)


NQ = NB * SEQ
NW = 32
QPW = SEQ // NW
CQ = 8
CR = CQ * TOPK
LANES = 16


NCHUNK = QPW // CQ


def _sc_body(v_hbm, idx_hbm, w_hbm, out_hbm, idx_v, w_v, rows_v, out_v,
             sems, osems):
    wid = lax.axis_index("s") * 2 + lax.axis_index("c")
    qbase = wid * QPW

    pltpu.sync_copy(idx_hbm.at[pl.ds(qbase * TOPK, QPW * TOPK)], idx_v)
    pltpu.sync_copy(w_hbm.at[pl.ds(qbase * TOPK, QPW * TOPK)], w_v)

    def issue(c):
        return pltpu.async_copy(v_hbm.at[idx_v.at[pl.ds(c * CR, CR)]],
                                rows_v.at[c % 2], sems.at[c % 2])

    cps = {0: issue(0)}
    ocps = {}
    for c in range(NCHUNK):
        p = c % 2
        if c + 1 < NCHUNK:
            cps[c + 1] = issue(c + 1)
        cps[c].wait()
        if c >= 2:
            ocps[c - 2].wait()
        for qi in range(CQ):
            wb = [
                plsc.load_gather(
                    w_v, [jnp.full((LANES,), c * CR + qi * TOPK + j,
                                   jnp.int32)])
                for j in range(TOPK)
            ]

            @pl.loop(0, DIM, step=LANES * 8)
            def _(d):
                for u in range(8):
                    sl = pl.ds(d + u * LANES, LANES)
                    acc = rows_v[p, qi * TOPK + 0, sl] * wb[0]
                    acc = acc + rows_v[p, qi * TOPK + 1, sl] * wb[1]
                    acc = acc + rows_v[p, qi * TOPK + 2, sl] * wb[2]
                    acc = acc + rows_v[p, qi * TOPK + 3, sl] * wb[3]
                    out_v[p, qi, sl] = acc

        ocps[c] = pltpu.async_copy(out_v.at[p],
                                   out_hbm.at[pl.ds(qbase + c * CQ, CQ)],
                                   osems.at[p])
    ocps[NCHUNK - 2].wait()
    ocps[NCHUNK - 1].wait()


def _sc_stage(v_flat, idx_flat, w_flat):
    mesh = plsc.VectorSubcoreMesh(core_axis_name="c", subcore_axis_name="s")
    cp = pltpu.CompilerParams()
    if "needs_layout_passes" in pltpu.CompilerParams.__dataclass_fields__:
        cp = dataclasses.replace(cp, needs_layout_passes=False)
    f = pl.kernel(
        _sc_body,
        out_type=jax.ShapeDtypeStruct((SEQ, DIM), jnp.float32),
        mesh=mesh,
        scratch_types=[
            pltpu.VMEM((QPW * TOPK,), jnp.int32),
            pltpu.VMEM((QPW * TOPK,), jnp.float32),
            pltpu.VMEM((2, CR, DIM), jnp.float32),
            pltpu.VMEM((2, CQ, DIM), jnp.float32),
            pltpu.SemaphoreType.DMA((2,)),
            pltpu.SemaphoreType.DMA((2,)),
        ],
        compiler_params=cp,
    )
    return f(v_flat, idx_flat, w_flat)


def kernel(x, Wq, bq, Wk, bk, Wv, bv):
    x2d = x.reshape(NB * SEQ, DIM)
    k, v = _proj_stage(x2d, Wk, bk, Wv, bv)
    wis = [_attn_stage(x2d, Wq, bq, k, b) for b in range(NB)]
    outs = [_sc_stage(v, idx.reshape(SEQ * TOPK), w.reshape(SEQ * TOPK))
            for (w, idx) in wis]
    return jnp.stack(outs).reshape(NB, SEQ, DIM)
